# baseline (speedup 1.0000x reference)
import functools

import jax
import jax.numpy as jnp
from jax import lax
from jax.experimental import pallas as pl
from jax.experimental.pallas import tpu as pltpu
from jax.experimental.pallas import tpu_sc as plsc

B = 16384
F = 32
NC = 2
NS = 16
NW = NC * NS
BPW = B // NW
CHUNK = 128
NCHUNK = BPW // CHUNK
GROUPS = BPW // 16


def _mf_body(uid_hbm, iid_hbm, uembT_hbm, ubias_hbm, iembT_hbm, ibias_hbm,
             out_hbm, uid_v, iid_v, ucols_v, icols_v, ub_v, ib_v, out_v, sem):
    wid = lax.axis_index("s") * NC + lax.axis_index("c")
    base = wid * BPW

    pltpu.sync_copy(uid_hbm.at[pl.ds(base, BPW)], uid_v)
    pltpu.sync_copy(iid_hbm.at[pl.ds(base, BPW)], iid_v)

    copies = []
    for c in range(NCHUNK):
        sl = pl.ds(c * CHUNK, CHUNK)
        usl = uid_v.at[sl]
        isl = iid_v.at[sl]
        copies.append(pltpu.async_copy(ubias_hbm.at[usl], ub_v.at[sl], sem))
        copies.append(pltpu.async_copy(ibias_hbm.at[isl], ib_v.at[sl], sem))
        for f in range(F):
            copies.append(pltpu.async_copy(
                uembT_hbm.at[f].at[usl], ucols_v.at[f].at[sl], sem))
            copies.append(pltpu.async_copy(
                iembT_hbm.at[f].at[isl], icols_v.at[f].at[sl], sem))
    for cp in copies:
        cp.wait()

    def group(g, carry):
        i0 = g * 16
        gsl = pl.ds(i0, 16)
        acc = ub_v[gsl] + ib_v[gsl]
        for f in range(F):
            acc = acc + ucols_v[f, gsl] * icols_v[f, gsl]
        out_v[gsl] = acc
        return carry

    lax.fori_loop(0, GROUPS, group, 0)

    pltpu.sync_copy(out_v, out_hbm.at[pl.ds(base, BPW)])


@jax.jit
def kernel(user_id, item_id, user_embedding, user_bias, item_embedding, item_bias):
    run = pl.kernel(
        _mf_body,
        out_type=jax.ShapeDtypeStruct((B,), jnp.float32),
        mesh=plsc.VectorSubcoreMesh(core_axis_name="c", subcore_axis_name="s"),
        compiler_params=pltpu.CompilerParams(
            needs_layout_passes=False, use_tc_tiling_on_sc=False),
        scratch_types=[
            pltpu.VMEM((BPW,), jnp.int32),
            pltpu.VMEM((BPW,), jnp.int32),
            pltpu.VMEM((F, BPW), jnp.float32),
            pltpu.VMEM((F, BPW), jnp.float32),
            pltpu.VMEM((BPW,), jnp.float32),
            pltpu.VMEM((BPW,), jnp.float32),
            pltpu.VMEM((BPW,), jnp.float32),
            pltpu.SemaphoreType.DMA,
        ],
    )
    return run(user_id, item_id, user_embedding.T, user_bias.reshape(-1),
               item_embedding.T, item_bias.reshape(-1))


# speedup vs baseline: 4.4270x; 4.4270x over previous
import functools

import jax
import jax.numpy as jnp
from jax import lax
from jax.experimental import pallas as pl
from jax.experimental.pallas import tpu as pltpu
from jax.experimental.pallas import tpu_sc as plsc

B = 16384
F = 32
NC = 2
NS = 16
NW = NC * NS
BPW = B // NW
CHUNK = 128
NCHUNK = BPW // CHUNK
GROUPS = BPW // 16


def _mf_body(uid_hbm, iid_hbm, uemb_hbm, ubias_hbm, iemb_hbm, ibias_hbm,
             out_hbm, uid_v, iid_v, urows_v, irows_v, ub_v, ib_v, out_v,
             tmp_v, sem):
    wid = lax.axis_index("s") * NC + lax.axis_index("c")
    base = wid * BPW

    pltpu.sync_copy(uid_hbm.at[pl.ds(base, BPW)], uid_v)
    pltpu.sync_copy(iid_hbm.at[pl.ds(base, BPW)], iid_v)

    copies = []
    for c in range(NCHUNK):
        sl = pl.ds(c * CHUNK, CHUNK)
        copies.append(pltpu.async_copy(uemb_hbm.at[uid_v.at[sl]], urows_v.at[sl], sem))
        copies.append(pltpu.async_copy(iemb_hbm.at[iid_v.at[sl]], irows_v.at[sl], sem))
        copies.append(pltpu.async_copy(ubias_hbm.at[uid_v.at[sl]], ub_v.at[sl], sem))
        copies.append(pltpu.async_copy(ibias_hbm.at[iid_v.at[sl]], ib_v.at[sl], sem))
    for cp in copies:
        cp.wait()

    lane16 = lax.iota(jnp.int32, 16) * 16

    def group(g, carry):
        b0 = g * 16
        for j in range(16):
            b = b0 + j
            s = (urows_v[b, pl.ds(0, 16)] * irows_v[b, pl.ds(0, 16)]
                 + urows_v[b, pl.ds(16, 16)] * irows_v[b, pl.ds(16, 16)])
            tmp_v[pl.ds(j * 16, 16)] = s
        acc = ub_v[pl.ds(b0, 16)] + ib_v[pl.ds(b0, 16)]
        for c in range(16):
            acc = acc + plsc.load_gather(tmp_v, [lane16 + c])
        out_v[pl.ds(b0, 16)] = acc
        return carry

    lax.fori_loop(0, GROUPS, group, 0)

    pltpu.sync_copy(out_v, out_hbm.at[pl.ds(base, BPW)])


@jax.jit
def kernel(user_id, item_id, user_embedding, user_bias, item_embedding, item_bias):
    run = pl.kernel(
        _mf_body,
        out_type=jax.ShapeDtypeStruct((B,), jnp.float32),
        mesh=plsc.VectorSubcoreMesh(core_axis_name="c", subcore_axis_name="s"),
        compiler_params=pltpu.CompilerParams(
            needs_layout_passes=False, use_tc_tiling_on_sc=False),
        scratch_types=[
            pltpu.VMEM((BPW,), jnp.int32),
            pltpu.VMEM((BPW,), jnp.int32),
            pltpu.VMEM((BPW, F), jnp.float32),
            pltpu.VMEM((BPW, F), jnp.float32),
            pltpu.VMEM((BPW,), jnp.float32),
            pltpu.VMEM((BPW,), jnp.float32),
            pltpu.VMEM((BPW,), jnp.float32),
            pltpu.VMEM((256,), jnp.float32),
            pltpu.SemaphoreType.DMA,
        ],
    )
    eye = jnp.eye(F, dtype=jnp.float32)
    return run(user_id, item_id, user_embedding @ eye, user_bias.reshape(-1),
               item_embedding @ eye, item_bias.reshape(-1))


# speedup vs baseline: 5.7994x; 1.3100x over previous
import functools

import jax
import jax.numpy as jnp
from jax import lax
from jax.experimental import pallas as pl
from jax.experimental.pallas import tpu as pltpu
from jax.experimental.pallas import tpu_sc as plsc

B = 16384
F = 32
NC = 2
NS = 16
NW = NC * NS
BPW = B // NW
CHUNK = 128
NCHUNK = BPW // CHUNK
GROUPS = BPW // 16


def _mf_body(uid_hbm, iid_hbm, uemb_hbm, ubias_hbm, iemb_hbm, ibias_hbm,
             out_hbm, uid_v, iid_v, urows_v, irows_v, ub_v, ib_v, out_v,
             tmp_v, sem):
    wid = lax.axis_index("s") * NC + lax.axis_index("c")
    base = wid * BPW

    pltpu.sync_copy(uid_hbm.at[pl.ds(base, BPW)], uid_v)
    pltpu.sync_copy(iid_hbm.at[pl.ds(base, BPW)], iid_v)

    copies = []
    for c in range(NCHUNK):
        sl = pl.ds(c * CHUNK, CHUNK)
        copies.append(pltpu.async_copy(uemb_hbm.at[uid_v.at[sl]], urows_v.at[sl], sem))
        copies.append(pltpu.async_copy(iemb_hbm.at[iid_v.at[sl]], irows_v.at[sl], sem))
        copies.append(pltpu.async_copy(ubias_hbm.at[uid_v.at[sl]], ub_v.at[sl], sem))
        copies.append(pltpu.async_copy(ibias_hbm.at[iid_v.at[sl]], ib_v.at[sl], sem))
    for cp in copies:
        cp.wait()

    lane16 = lax.iota(jnp.int32, 16) * 16

    def group(g, carry):
        b0 = g * 16
        for j in range(16):
            b = b0 + j
            s = (urows_v[b, pl.ds(0, 16)] * irows_v[b, pl.ds(0, 16)]
                 + urows_v[b, pl.ds(16, 16)] * irows_v[b, pl.ds(16, 16)])
            tmp_v[pl.ds(j * 16, 16)] = s
        acc = ub_v[pl.ds(b0, 16)] + ib_v[pl.ds(b0, 16)]
        for c in range(16):
            acc = acc + plsc.load_gather(tmp_v, [lane16 + c])
        out_v[pl.ds(b0, 16)] = acc
        return carry

    lax.fori_loop(0, GROUPS, group, 0)

    pltpu.sync_copy(out_v, out_hbm.at[pl.ds(base, BPW)])


@jax.jit
def kernel(user_id, item_id, user_embedding, user_bias, item_embedding, item_bias):
    run = pl.kernel(
        _mf_body,
        out_type=jax.ShapeDtypeStruct((B,), jnp.float32),
        mesh=plsc.VectorSubcoreMesh(core_axis_name="c", subcore_axis_name="s"),
        compiler_params=pltpu.CompilerParams(
            needs_layout_passes=False, use_tc_tiling_on_sc=False),
        scratch_types=[
            pltpu.VMEM((BPW,), jnp.int32),
            pltpu.VMEM((BPW,), jnp.int32),
            pltpu.VMEM((BPW, F), jnp.float32),
            pltpu.VMEM((BPW, F), jnp.float32),
            pltpu.VMEM((BPW,), jnp.float32),
            pltpu.VMEM((BPW,), jnp.float32),
            pltpu.VMEM((BPW,), jnp.float32),
            pltpu.VMEM((256,), jnp.float32),
            pltpu.SemaphoreType.DMA,
        ],
    )
    return run(user_id, item_id, user_embedding, user_bias.reshape(-1),
               item_embedding, item_bias.reshape(-1))
